# Initial kernel scaffold; baseline (speedup 1.0000x reference)
import jax
import jax.numpy as jnp
from jax.experimental import pallas as pl


def kernel(user_emb, item_emb, edge_index):
    raise NotImplementedError("write your pallas kernel here")


# speedup vs baseline: 14.3516x; 14.3516x over previous
import jax
import jax.numpy as jnp
from jax import lax
from jax.experimental import pallas as pl
from jax.experimental.pallas import tpu as pltpu
from jax.experimental.pallas import tpu_sc as plsc

NU = 25000
D = 64
HD = 32
E = 400000
NC = 2
NS = 16
HALF = 25088
ROWS_PER_TILE = HALF // NS
TROWS = NC * HALF
CHUNK = 128
BLK_CHUNKS = 4
EP = 401408
CHUNKS = EP // CHUNK
CHUNKS_PER_TILE = CHUNKS // NS
NBLOCKS = CHUNKS_PER_TILE // BLK_CHUNKS

_MESH = plsc.VectorSubcoreMesh(
    core_axis_name="c", subcore_axis_name="s", num_cores=NC, num_subcores=NS)
_SC_PARAMS = pltpu.CompilerParams(use_tc_tiling_on_sc=False)


def _prop_body(t0, t1, gidx, lidx, zeros32, o0, o1, gidx_v, lidx_v, rows_v,
               sem, acc_sh):
  c = lax.axis_index("c")
  s = lax.axis_index("s")
  row0 = s * ROWS_PER_TILE
  chunk0 = s * CHUNKS_PER_TILE
  for tab, out in ((t0, o0), (t1, o1)):
    pltpu.sync_copy(zeros32, acc_sh.at[pl.ds(row0, ROWS_PER_TILE)])
    plsc.subcore_barrier()

    def blk(b, carry, tab=tab):
      ch = chunk0 + b * BLK_CHUNKS
      pltpu.sync_copy(gidx.at[c, pl.ds(ch, BLK_CHUNKS)], gidx_v)
      pltpu.sync_copy(lidx.at[c, pl.ds(ch, BLK_CHUNKS)], lidx_v)
      descs = [
          pltpu.async_copy(tab.at[gidx_v.at[j]], rows_v.at[j], sem)
          for j in range(BLK_CHUNKS)
      ]
      for dd in descs:
        dd.wait()
      for j in range(BLK_CHUNKS):
        pltpu.sync_copy(rows_v.at[j], acc_sh.at[lidx_v.at[j]], add=True)
      return carry

    lax.fori_loop(0, NBLOCKS, blk, 0)
    plsc.subcore_barrier()
    pltpu.sync_copy(acc_sh.at[pl.ds(row0, ROWS_PER_TILE)],
                    out.at[c, pl.ds(row0, ROWS_PER_TILE)])


_prop = pl.kernel(
    _prop_body,
    out_type=(jax.ShapeDtypeStruct((NC, HALF, HD), jnp.float32),
              jax.ShapeDtypeStruct((NC, HALF, HD), jnp.float32)),
    mesh=_MESH,
    compiler_params=_SC_PARAMS,
    scratch_types=[
        pltpu.VMEM((BLK_CHUNKS, CHUNK), jnp.int32),
        pltpu.VMEM((BLK_CHUNKS, CHUNK), jnp.int32),
        pltpu.VMEM((BLK_CHUNKS, CHUNK, HD), jnp.float32),
        pltpu.SemaphoreType.DMA,
        pltpu.VMEM_SHARED((HALF, HD), jnp.float32),
    ],
)


def _deg_body(lidx, ones, zeros1, out, lidx_v, ones_v, deg_sh):
  c = lax.axis_index("c")
  s = lax.axis_index("s")
  row0 = s * ROWS_PER_TILE
  pltpu.sync_copy(zeros1, deg_sh.at[pl.ds(row0, ROWS_PER_TILE)])
  pltpu.sync_copy(ones, ones_v)
  plsc.subcore_barrier()
  chunk0 = s * CHUNKS_PER_TILE

  def blk(b, carry):
    ch = chunk0 + b * BLK_CHUNKS
    pltpu.sync_copy(lidx.at[c, pl.ds(ch, BLK_CHUNKS)], lidx_v)
    for j in range(BLK_CHUNKS):
      pltpu.sync_copy(ones_v, deg_sh.at[lidx_v.at[j]], add=True)
    return carry

  lax.fori_loop(0, NBLOCKS, blk, 0)
  plsc.subcore_barrier()
  pltpu.sync_copy(deg_sh.at[pl.ds(row0, ROWS_PER_TILE)],
                  out.at[c, pl.ds(row0, ROWS_PER_TILE)])


_deg = pl.kernel(
    _deg_body,
    out_type=jax.ShapeDtypeStruct((NC, HALF), jnp.float32),
    mesh=_MESH,
    compiler_params=_SC_PARAMS,
    scratch_types=[
        pltpu.VMEM((BLK_CHUNKS, CHUNK), jnp.int32),
        pltpu.VMEM((CHUNK,), jnp.float32),
        pltpu.VMEM_SHARED((HALF,), jnp.float32),
    ],
)


def _scale_rsqrt_body(x0_ref, x1_ref, d_ref, o0_ref, o1_ref):
  sc = lax.rsqrt(jnp.maximum(d_ref[...], 1.0))
  o0_ref[...] = (x0_ref[...] * sc)[0]
  o1_ref[...] = (x1_ref[...] * sc)[0]


def _scale_inv_body(x0_ref, x1_ref, d_ref, o0_ref, o1_ref):
  sc = 1.0 / jnp.maximum(d_ref[...], 1.0)
  o0_ref[...] = (x0_ref[...] * sc)[0]
  o1_ref[...] = (x1_ref[...] * sc)[0]


def _make_scale(body):
  return pl.pallas_call(
      body,
      out_shape=(jax.ShapeDtypeStruct((TROWS, HD), jnp.float32),
                 jax.ShapeDtypeStruct((TROWS, HD), jnp.float32)),
      grid=(NC, NS),
      in_specs=[
          pl.BlockSpec((1, ROWS_PER_TILE, HD), lambda c, b: (c, b, 0)),
          pl.BlockSpec((1, ROWS_PER_TILE, HD), lambda c, b: (c, b, 0)),
          pl.BlockSpec((1, ROWS_PER_TILE, 1), lambda c, b: (c, b, 0)),
      ],
      out_specs=(
          pl.BlockSpec((ROWS_PER_TILE, HD), lambda c, b: (c * NS + b, 0)),
          pl.BlockSpec((ROWS_PER_TILE, HD), lambda c, b: (c * NS + b, 0)),
      ),
  )


_scale_rsqrt = _make_scale(_scale_rsqrt_body)
_scale_inv = _make_scale(_scale_inv_body)

_BR = 1000
_NB = NU // _BR


def _final_body(e0_ref, a0_ref, a1_ref, b0_ref, b1_ref, c0_ref, c1_ref,
                d_ref, o_ref):
  dis = lax.rsqrt(jnp.maximum(d_ref[...], 1.0))
  s0 = ((a0_ref[...] + b0_ref[...] + c0_ref[...]) * dis)[0]
  s1 = ((a1_ref[...] + b1_ref[...] + c1_ref[...]) * dis)[0]
  o_ref[...] = 0.25 * (e0_ref[...] + jnp.concatenate([s0, s1], axis=-1))


_final = pl.pallas_call(
    _final_body,
    out_shape=jax.ShapeDtypeStruct((2 * NU, D), jnp.float32),
    grid=(NC, _NB),
    in_specs=[
        pl.BlockSpec((_BR, D), lambda c, b: (c * _NB + b, 0)),
        pl.BlockSpec((1, _BR, HD), lambda c, b: (c, b, 0)),
        pl.BlockSpec((1, _BR, HD), lambda c, b: (c, b, 0)),
        pl.BlockSpec((1, _BR, HD), lambda c, b: (c, b, 0)),
        pl.BlockSpec((1, _BR, HD), lambda c, b: (c, b, 0)),
        pl.BlockSpec((1, _BR, HD), lambda c, b: (c, b, 0)),
        pl.BlockSpec((1, _BR, HD), lambda c, b: (c, b, 0)),
        pl.BlockSpec((1, _BR, 1), lambda c, b: (c, b, 0)),
    ],
    out_specs=pl.BlockSpec((_BR, D), lambda c, b: (c * _NB + b, 0)),
)


def kernel(user_emb, item_emb, edge_index):
  src = edge_index[0].astype(jnp.int32)
  dst = edge_index[1].astype(jnp.int32)

  npad = EP - E
  padg = jnp.zeros((npad,), jnp.int32)
  padl = jnp.full((npad,), NU, jnp.int32)
  g_u = jnp.concatenate([dst + HALF, padg])
  l_u = jnp.concatenate([src, padl])
  g_i = jnp.concatenate([src, padg])
  l_i = jnp.concatenate([dst, padl])
  gidx = jnp.stack([g_u, g_i]).reshape(NC, CHUNKS, CHUNK)
  lidx = jnp.stack([l_u, l_i]).reshape(NC, CHUNKS, CHUNK)

  zeros32 = jnp.zeros((ROWS_PER_TILE, HD), jnp.float32)
  zeros1 = jnp.zeros((ROWS_PER_TILE,), jnp.float32)
  ones = jnp.ones((CHUNK,), jnp.float32)

  deg = _deg(lidx, ones, zeros1)
  deg3 = deg[:, :, None]

  pad_rows = jnp.zeros((HALF - NU, HD), jnp.float32)
  e0p0 = jnp.stack([
      jnp.concatenate([user_emb[:, :HD], pad_rows], axis=0),
      jnp.concatenate([item_emb[:, :HD], pad_rows], axis=0),
  ])
  e0p1 = jnp.stack([
      jnp.concatenate([user_emb[:, HD:], pad_rows], axis=0),
      jnp.concatenate([item_emb[:, HD:], pad_rows], axis=0),
  ])

  g0, g1 = _scale_rsqrt(e0p0, e0p1, deg3)
  hs = []
  for k in range(3):
    h0, h1 = _prop(g0, g1, gidx, lidx, zeros32)
    hs.append((h0, h1))
    if k < 2:
      g0, g1 = _scale_inv(h0, h1, deg3)

  e0 = jnp.concatenate([user_emb, item_emb], axis=0)
  return _final(e0, hs[0][0], hs[0][1], hs[1][0], hs[1][1], hs[2][0],
                hs[2][1], deg3)


# speedup vs baseline: 19.5826x; 1.3645x over previous
import jax
import jax.numpy as jnp
from jax import lax
from jax.experimental import pallas as pl
from jax.experimental.pallas import tpu as pltpu
from jax.experimental.pallas import tpu_sc as plsc

NU = 25000
D = 64
HD = 32
E = 400000
NC = 2
NS = 16
HALF = 25088
ROWS_PER_TILE = HALF // NS
TROWS = NC * HALF
CHUNK = 128
BLK_CHUNKS = 4
EP = 401408
CHUNKS = EP // CHUNK
CHUNKS_PER_TILE = CHUNKS // NS
NBLOCKS = CHUNKS_PER_TILE // BLK_CHUNKS
ITER_CHUNKS = 16
NITER = CHUNKS_PER_TILE // ITER_CHUNKS
TAIL_CHUNKS = CHUNKS_PER_TILE - NITER * ITER_CHUNKS

_MESH = plsc.VectorSubcoreMesh(
    core_axis_name="c", subcore_axis_name="s", num_cores=NC, num_subcores=NS)
_SC_PARAMS = pltpu.CompilerParams(use_tc_tiling_on_sc=False)


def _prop_body(t0, t1, gidx, lidx, zeros32, o0, o1, gidx_v, lidx_v, rows_v,
               gsem, ssem, acc_sh):
  c = lax.axis_index("c")
  s = lax.axis_index("s")
  row0 = s * ROWS_PER_TILE
  chunk0 = s * CHUNKS_PER_TILE

  def run_block(tab, ch, n):
    pltpu.sync_copy(gidx.at[c, pl.ds(ch, n)], gidx_v.at[pl.ds(0, n)])
    pltpu.sync_copy(lidx.at[c, pl.ds(ch, n)], lidx_v.at[pl.ds(0, n)])
    gd = [
        pltpu.async_copy(tab.at[gidx_v.at[j]], rows_v.at[j], gsem)
        for j in range(n)
    ]
    sd = []
    for j in range(n):
      gd[j].wait()
      sd.append(
          pltpu.async_copy(rows_v.at[j], acc_sh.at[lidx_v.at[j]], ssem,
                           add=True))
    for dd in sd:
      dd.wait()

  for tab, out in ((t0, o0), (t1, o1)):
    pltpu.sync_copy(zeros32, acc_sh.at[pl.ds(row0, ROWS_PER_TILE)])
    plsc.subcore_barrier()

    def blk(it, carry, tab=tab):
      run_block(tab, chunk0 + it * ITER_CHUNKS, ITER_CHUNKS)
      return carry

    lax.fori_loop(0, NITER, blk, 0)
    run_block(tab, chunk0 + NITER * ITER_CHUNKS, TAIL_CHUNKS)
    plsc.subcore_barrier()
    pltpu.sync_copy(acc_sh.at[pl.ds(row0, ROWS_PER_TILE)],
                    out.at[c, pl.ds(row0, ROWS_PER_TILE)])


_prop = pl.kernel(
    _prop_body,
    out_type=(jax.ShapeDtypeStruct((NC, HALF, HD), jnp.float32),
              jax.ShapeDtypeStruct((NC, HALF, HD), jnp.float32)),
    mesh=_MESH,
    compiler_params=_SC_PARAMS,
    scratch_types=[
        pltpu.VMEM((ITER_CHUNKS, CHUNK), jnp.int32),
        pltpu.VMEM((ITER_CHUNKS, CHUNK), jnp.int32),
        pltpu.VMEM((ITER_CHUNKS, CHUNK, HD), jnp.float32),
        pltpu.SemaphoreType.DMA,
        pltpu.SemaphoreType.DMA,
        pltpu.VMEM_SHARED((HALF, HD), jnp.float32),
    ],
)


def _deg_body(lidx, ones, zeros1, out, lidx_v, ones_v, deg_sh):
  c = lax.axis_index("c")
  s = lax.axis_index("s")
  row0 = s * ROWS_PER_TILE
  pltpu.sync_copy(zeros1, deg_sh.at[pl.ds(row0, ROWS_PER_TILE)])
  pltpu.sync_copy(ones, ones_v)
  plsc.subcore_barrier()
  chunk0 = s * CHUNKS_PER_TILE

  def blk(b, carry):
    ch = chunk0 + b * BLK_CHUNKS
    pltpu.sync_copy(lidx.at[c, pl.ds(ch, BLK_CHUNKS)], lidx_v)
    for j in range(BLK_CHUNKS):
      pltpu.sync_copy(ones_v, deg_sh.at[lidx_v.at[j]], add=True)
    return carry

  lax.fori_loop(0, NBLOCKS, blk, 0)
  plsc.subcore_barrier()
  pltpu.sync_copy(deg_sh.at[pl.ds(row0, ROWS_PER_TILE)],
                  out.at[c, pl.ds(row0, ROWS_PER_TILE)])


_deg = pl.kernel(
    _deg_body,
    out_type=jax.ShapeDtypeStruct((NC, HALF), jnp.float32),
    mesh=_MESH,
    compiler_params=_SC_PARAMS,
    scratch_types=[
        pltpu.VMEM((BLK_CHUNKS, CHUNK), jnp.int32),
        pltpu.VMEM((CHUNK,), jnp.float32),
        pltpu.VMEM_SHARED((HALF,), jnp.float32),
    ],
)


def _scale_rsqrt_body(x0_ref, x1_ref, d_ref, o0_ref, o1_ref):
  sc = lax.rsqrt(jnp.maximum(d_ref[...], 1.0))
  o0_ref[...] = (x0_ref[...] * sc)[0]
  o1_ref[...] = (x1_ref[...] * sc)[0]


def _scale_inv_body(x0_ref, x1_ref, d_ref, o0_ref, o1_ref):
  sc = 1.0 / jnp.maximum(d_ref[...], 1.0)
  o0_ref[...] = (x0_ref[...] * sc)[0]
  o1_ref[...] = (x1_ref[...] * sc)[0]


def _make_scale(body):
  return pl.pallas_call(
      body,
      out_shape=(jax.ShapeDtypeStruct((TROWS, HD), jnp.float32),
                 jax.ShapeDtypeStruct((TROWS, HD), jnp.float32)),
      grid=(NC, NS),
      in_specs=[
          pl.BlockSpec((1, ROWS_PER_TILE, HD), lambda c, b: (c, b, 0)),
          pl.BlockSpec((1, ROWS_PER_TILE, HD), lambda c, b: (c, b, 0)),
          pl.BlockSpec((1, ROWS_PER_TILE, 1), lambda c, b: (c, b, 0)),
      ],
      out_specs=(
          pl.BlockSpec((ROWS_PER_TILE, HD), lambda c, b: (c * NS + b, 0)),
          pl.BlockSpec((ROWS_PER_TILE, HD), lambda c, b: (c * NS + b, 0)),
      ),
  )


_scale_rsqrt = _make_scale(_scale_rsqrt_body)
_scale_inv = _make_scale(_scale_inv_body)

_BR = 1000
_NB = NU // _BR


def _final_body(e0_ref, a0_ref, a1_ref, b0_ref, b1_ref, c0_ref, c1_ref,
                d_ref, o_ref):
  dis = lax.rsqrt(jnp.maximum(d_ref[...], 1.0))
  s0 = ((a0_ref[...] + b0_ref[...] + c0_ref[...]) * dis)[0]
  s1 = ((a1_ref[...] + b1_ref[...] + c1_ref[...]) * dis)[0]
  o_ref[...] = 0.25 * (e0_ref[...] + jnp.concatenate([s0, s1], axis=-1))


_final = pl.pallas_call(
    _final_body,
    out_shape=jax.ShapeDtypeStruct((2 * NU, D), jnp.float32),
    grid=(NC, _NB),
    in_specs=[
        pl.BlockSpec((_BR, D), lambda c, b: (c * _NB + b, 0)),
        pl.BlockSpec((1, _BR, HD), lambda c, b: (c, b, 0)),
        pl.BlockSpec((1, _BR, HD), lambda c, b: (c, b, 0)),
        pl.BlockSpec((1, _BR, HD), lambda c, b: (c, b, 0)),
        pl.BlockSpec((1, _BR, HD), lambda c, b: (c, b, 0)),
        pl.BlockSpec((1, _BR, HD), lambda c, b: (c, b, 0)),
        pl.BlockSpec((1, _BR, HD), lambda c, b: (c, b, 0)),
        pl.BlockSpec((1, _BR, 1), lambda c, b: (c, b, 0)),
    ],
    out_specs=pl.BlockSpec((_BR, D), lambda c, b: (c * _NB + b, 0)),
)


def kernel(user_emb, item_emb, edge_index):
  src = edge_index[0].astype(jnp.int32)
  dst = edge_index[1].astype(jnp.int32)

  npad = EP - E
  padg = jnp.zeros((npad,), jnp.int32)
  padl = jnp.full((npad,), NU, jnp.int32)
  g_u = jnp.concatenate([dst + HALF, padg])
  l_u = jnp.concatenate([src, padl])
  g_i = jnp.concatenate([src, padg])
  l_i = jnp.concatenate([dst, padl])
  gidx = jnp.stack([g_u, g_i]).reshape(NC, CHUNKS, CHUNK)
  lidx = jnp.stack([l_u, l_i]).reshape(NC, CHUNKS, CHUNK)

  zeros32 = jnp.zeros((ROWS_PER_TILE, HD), jnp.float32)
  zeros1 = jnp.zeros((ROWS_PER_TILE,), jnp.float32)
  ones = jnp.ones((CHUNK,), jnp.float32)

  deg = _deg(lidx, ones, zeros1)
  deg3 = deg[:, :, None]

  pad_rows = jnp.zeros((HALF - NU, HD), jnp.float32)
  e0p0 = jnp.stack([
      jnp.concatenate([user_emb[:, :HD], pad_rows], axis=0),
      jnp.concatenate([item_emb[:, :HD], pad_rows], axis=0),
  ])
  e0p1 = jnp.stack([
      jnp.concatenate([user_emb[:, HD:], pad_rows], axis=0),
      jnp.concatenate([item_emb[:, HD:], pad_rows], axis=0),
  ])

  g0, g1 = _scale_rsqrt(e0p0, e0p1, deg3)
  hs = []
  for k in range(3):
    h0, h1 = _prop(g0, g1, gidx, lidx, zeros32)
    hs.append((h0, h1))
    if k < 2:
      g0, g1 = _scale_inv(h0, h1, deg3)

  e0 = jnp.concatenate([user_emb, item_emb], axis=0)
  return _final(e0, hs[0][0], hs[0][1], hs[1][0], hs[1][1], hs[2][0],
                hs[2][1], deg3)
